# baseline (speedup 1.0000x reference)
import jax
import jax.numpy as jnp
from jax import lax
from jax.experimental import pallas as pl
from jax.experimental.pallas import tpu as pltpu
from jax.experimental.pallas import tpu_sc as plsc

NUM_CORES = 2
NUM_SUBCORES = 16
NW = NUM_CORES * NUM_SUBCORES
DIM = 64
BATCH = 16384
BPW = BATCH // NW
CH = 32
NCH = BPW // CH


def _body(h_hbm, r_hbm, t_hbm, ent_hbm, rel_hbm, out_hbm,
          hidx_v, ridx_v, tidx_v, hrow_v, rrow_v, trow_v, out_v, sem):
    cid = lax.axis_index("c")
    sid = lax.axis_index("s")
    wid = sid * NUM_CORES + cid
    base = wid * BPW

    pltpu.sync_copy(h_hbm.at[pl.ds(base, BPW)], hidx_v)
    pltpu.sync_copy(r_hbm.at[pl.ds(base, BPW)], ridx_v)
    pltpu.sync_copy(t_hbm.at[pl.ds(base, BPW)], tidx_v)

    lane = lax.iota(jnp.int32, 16)
    zero16 = jnp.zeros((16,), jnp.int32)

    def chunk(g, _):
        row0 = g * CH
        copies = []
        for v in range(CH // 16):
            hv = hidx_v[pl.ds(row0 + v * 16, 16)]
            rv = ridx_v[pl.ds(row0 + v * 16, 16)]
            tv = tidx_v[pl.ds(row0 + v * 16, 16)]
            for j in range(16):
                i = v * 16 + j
                copies.append(pltpu.async_copy(
                    ent_hbm.at[jnp.sum(jnp.where(lane == j, hv, zero16))],
                    hrow_v.at[i], sem))
                copies.append(pltpu.async_copy(
                    rel_hbm.at[jnp.sum(jnp.where(lane == j, rv, zero16))],
                    rrow_v.at[i], sem))
                copies.append(pltpu.async_copy(
                    ent_hbm.at[jnp.sum(jnp.where(lane == j, tv, zero16))],
                    trow_v.at[i], sem))
        for c in copies:
            c.wait()

        for b in range(CH // 16):
            acc = jnp.zeros((16,), jnp.float32)
            for j in range(16):
                rj = b * 16 + j
                s = jnp.zeros((16,), jnp.float32)
                for k in range(DIM // 32):
                    h32 = plsc.bitcast(hrow_v[rj, pl.ds(k * 16, 16)], jnp.bfloat16)
                    t32 = plsc.bitcast(trow_v[rj, pl.ds(k * 16, 16)], jnp.bfloat16)
                    he, ho = plsc.unpack(h32, format=plsc.PackFormat.INTERLEAVED)
                    te, to = plsc.unpack(t32, format=plsc.PackFormat.INTERLEAVED)
                    re = rrow_v[rj, pl.ds(k * 32, 16)]
                    ro = rrow_v[rj, pl.ds(k * 32 + 16, 16)]
                    s = s + jnp.abs(he + re - te) + jnp.abs(ho + ro - to)
                tot = jnp.sum(s)
                acc = jnp.where(lane == j, -tot, acc)
            out_v[pl.ds(row0 + b * 16, 16)] = acc
        return 0

    lax.fori_loop(0, NCH, chunk, 0)

    pltpu.sync_copy(out_v, out_hbm.at[pl.ds(base, BPW)])


@jax.jit
def kernel(h, r, t, ent_weight, rel_weight):
    h1 = h.astype(jnp.int32)
    r1 = r.astype(jnp.int32)
    t1 = t.astype(jnp.int32)
    ent_bf = lax.bitcast_convert_type(
        ent_weight.astype(jnp.bfloat16).reshape(1000000, DIM // 2, 2),
        jnp.int32)
    rel_perm = (rel_weight.reshape(1000, 2, 16, 2)
                .transpose(0, 1, 3, 2)
                .reshape(1000, DIM))

    run = pl.kernel(
        _body,
        out_type=jax.ShapeDtypeStruct((BATCH,), jnp.float32),
        mesh=plsc.VectorSubcoreMesh(core_axis_name="c", subcore_axis_name="s"),
        compiler_params=pltpu.CompilerParams(needs_layout_passes=False),
        scratch_types=[
            pltpu.VMEM((BPW,), jnp.int32),
            pltpu.VMEM((BPW,), jnp.int32),
            pltpu.VMEM((BPW,), jnp.int32),
            pltpu.VMEM((CH, DIM // 2), jnp.int32),
            pltpu.VMEM((CH, DIM), jnp.float32),
            pltpu.VMEM((CH, DIM // 2), jnp.int32),
            pltpu.VMEM((BPW,), jnp.float32),
            pltpu.SemaphoreType.DMA,
        ],
    )
    return run(h1, r1, t1, ent_bf, rel_perm)


# speedup vs baseline: 4.0090x; 4.0090x over previous
import jax
import jax.numpy as jnp
from jax import lax
from jax.experimental import pallas as pl
from jax.experimental.pallas import tpu as pltpu
from jax.experimental.pallas import tpu_sc as plsc

NUM_CORES = 2
NUM_SUBCORES = 16
NW = NUM_CORES * NUM_SUBCORES
DIM = 64
BATCH = 16384
BPW = BATCH // NW
CH = 32
NCH = BPW // CH


def _body(h_hbm, r_hbm, t_hbm, ent_hbm, rel_hbm, out_hbm,
          hidx_v, ridx_v, tidx_v, hrow_v, rrow_v, trow_v, out_v, sem):
    cid = lax.axis_index("c")
    sid = lax.axis_index("s")
    wid = sid * NUM_CORES + cid
    base = wid * BPW

    pltpu.sync_copy(h_hbm.at[pl.ds(base, BPW)], hidx_v)
    pltpu.sync_copy(r_hbm.at[pl.ds(base, BPW)], ridx_v)
    pltpu.sync_copy(t_hbm.at[pl.ds(base, BPW)], tidx_v)

    lane = lax.iota(jnp.int32, 16)
    zero16 = jnp.zeros((16,), jnp.int32)

    def chunk(g, _):
        row0 = g * CH
        copies = []
        for v in range(CH // 16):
            hv = hidx_v[pl.ds(row0 + v * 16, 16)]
            rv = ridx_v[pl.ds(row0 + v * 16, 16)]
            tv = tidx_v[pl.ds(row0 + v * 16, 16)]
            for j in range(16):
                i = v * 16 + j
                copies.append(pltpu.async_copy(
                    ent_hbm.at[jnp.sum(jnp.where(lane == j, hv, zero16))],
                    hrow_v.at[i], sem))
                copies.append(pltpu.async_copy(
                    rel_hbm.at[jnp.sum(jnp.where(lane == j, rv, zero16))],
                    rrow_v.at[i], sem))
                copies.append(pltpu.async_copy(
                    ent_hbm.at[jnp.sum(jnp.where(lane == j, tv, zero16))],
                    trow_v.at[i], sem))
        for c in copies:
            c.wait()

        for b in range(CH // 16):
            acc = jnp.zeros((16,), jnp.float32)
            for j in range(16):
                rj = b * 16 + j
                s = jnp.zeros((16,), jnp.float32)
                for k in range(DIM // 16):
                    sl = pl.ds(k * 16, 16)
                    s = s + jnp.abs(hrow_v[rj, sl] + rrow_v[rj, sl]
                                    - trow_v[rj, sl])
                tot = jnp.sum(s)
                acc = jnp.where(lane == j, -tot, acc)
            out_v[pl.ds(row0 + b * 16, 16)] = acc
        return 0

    lax.fori_loop(0, NCH, chunk, 0)

    pltpu.sync_copy(out_v, out_hbm.at[pl.ds(base, BPW)])


@jax.jit
def kernel(h, r, t, ent_weight, rel_weight):
    h1 = h.astype(jnp.int32)
    r1 = r.astype(jnp.int32)
    t1 = t.astype(jnp.int32)

    run = pl.kernel(
        _body,
        out_type=jax.ShapeDtypeStruct((BATCH,), jnp.float32),
        mesh=plsc.VectorSubcoreMesh(core_axis_name="c", subcore_axis_name="s"),
        compiler_params=pltpu.CompilerParams(needs_layout_passes=False),
        scratch_types=[
            pltpu.VMEM((BPW,), jnp.int32),
            pltpu.VMEM((BPW,), jnp.int32),
            pltpu.VMEM((BPW,), jnp.int32),
            pltpu.VMEM((CH, DIM), jnp.float32),
            pltpu.VMEM((CH, DIM), jnp.float32),
            pltpu.VMEM((CH, DIM), jnp.float32),
            pltpu.VMEM((BPW,), jnp.float32),
            pltpu.SemaphoreType.DMA,
        ],
    )
    return run(h1, r1, t1, ent_weight, rel_weight)
